# baseline (speedup 1.0000x reference)
import functools

import jax
import jax.numpy as jnp
import numpy as np
from jax.experimental import pallas as pl
from jax.experimental.pallas import tpu as pltpu

_FMAP_DIMS = [(100, 100), (50, 50), (25, 25), (13, 13), (7, 7)]
_OBJ_SCALES = [0.1, 0.2, 0.4, 0.6, 0.8]
_N_CANDIDATES = 11
_GAMMA = 2.5
_ALPHA = 0.25
_N_CLASSES = 8
_N_OB = 8

_SIZES = [h * w for h, w in _FMAP_DIMS]
_BOUNDS = np.concatenate([[0], np.cumsum(_SIZES)]).tolist()
_N_PRIORS = _BOUNDS[-1]


def _make_priors_cxcy():
    out = []
    for (h, w), s in zip(_FMAP_DIMS, _OBJ_SCALES):
        cy, cx = jnp.meshgrid((jnp.arange(h) + 0.5) / h,
                              (jnp.arange(w) + 0.5) / w, indexing="ij")
        p = jnp.stack([cx.reshape(-1), cy.reshape(-1),
                       jnp.full((h * w,), s, dtype=jnp.float32),
                       jnp.full((h * w,), s, dtype=jnp.float32)], axis=1)
        out.append(jnp.clip(p, 0.0, 1.0))
    return jnp.concatenate(out, axis=0)


def _log_sigmoid(x):
    return jnp.minimum(x, 0.0) - jnp.log1p(jnp.exp(-jnp.abs(x)))


def _pow_gamma(p):
    return p * p * jnp.sqrt(p)


def _smooth_l1(x):
    ax = jnp.abs(x)
    return jnp.where(ax < 1.0, 0.5 * x * x, ax - 0.5)


def _loss_body(locs_ref, scores_ref, boxes_ref, labels_ref,
               pcT_ref, pxyT_ref, pc_ref, out_ref):
    scores = scores_ref[0]
    gt_xy = boxes_ref[0]
    labels_row = labels_ref[0]

    s = scores
    ea = jnp.exp(-jnp.abs(s))
    l1p = jnp.log1p(ea)
    ls_pos = jnp.minimum(s, 0.0) - l1p
    ls_neg = jnp.minimum(-s, 0.0) - l1p
    term2 = jnp.exp(_GAMMA * ls_pos) * ls_neg
    neg_sum = -(1.0 - _ALPHA) * jnp.sum(term2)

    gt_cx = (gt_xy[:, 0:1] + gt_xy[:, 2:3]) * 0.5
    gt_cy = (gt_xy[:, 1:2] + gt_xy[:, 3:4]) * 0.5
    area_g = (gt_xy[:, 2:3] - gt_xy[:, 0:1]) * (gt_xy[:, 3:4] - gt_xy[:, 1:2])

    per_level = []
    all_ov_cols = []
    for lv in range(len(_FMAP_DIMS)):
        s0, s1 = _BOUNDS[lv], _BOUNDS[lv + 1]
        px = pcT_ref[0:1, s0:s1]
        py = pcT_ref[1:2, s0:s1]
        x1 = pxyT_ref[0:1, s0:s1]
        y1 = pxyT_ref[1:2, s0:s1]
        x2 = pxyT_ref[2:3, s0:s1]
        y2 = pxyT_ref[3:4, s0:s1]
        P = s1 - s0

        dx = gt_cx - px
        dy = gt_cy - py
        dist = jnp.sqrt(dx * dx + dy * dy)

        iw = jnp.clip(jnp.minimum(gt_xy[:, 2:3], x2) -
                      jnp.maximum(gt_xy[:, 0:1], x1), 0.0, None)
        ih = jnp.clip(jnp.minimum(gt_xy[:, 3:4], y2) -
                      jnp.maximum(gt_xy[:, 1:2], y1), 0.0, None)
        inter = iw * ih
        area_p = (x2 - x1) * (y2 - y1)
        ovl = inter / (area_g + area_p - inter)

        inside = jnp.where((gt_xy[:, 0:1] <= px) & (px <= gt_xy[:, 2:3]) &
                           (gt_xy[:, 1:2] <= py) & (py <= gt_xy[:, 3:4]),
                           jnp.int32(1), jnp.int32(0))

        iota = jax.lax.broadcasted_iota(jnp.int32, (_N_OB, P), 1)
        d = dist
        ov_cols, in_cols, idx_cols = [], [], []
        for _ in range(_N_CANDIDATES):
            mval = jnp.min(d, axis=1, keepdims=True)
            midx = jnp.min(jnp.where(d == mval, iota, P),
                           axis=1, keepdims=True)
            onehot = iota == midx
            ov_c = jnp.sum(jnp.where(onehot, ovl, 0.0),
                           axis=1, keepdims=True)
            in_c = jnp.max(jnp.where(onehot, inside, 0),
                           axis=1, keepdims=True)
            d = jnp.where(onehot, 1e30, d)
            ov_cols.append(ov_c)
            in_cols.append(in_c)
            idx_cols.append(midx)
        per_level.append((ov_cols, in_cols, idx_cols))
        all_ov_cols.extend(ov_cols)

    ov_all = jnp.concatenate(all_ov_cols, axis=1)
    n_cand = ov_all.shape[1]
    mean = jnp.mean(ov_all, axis=1, keepdims=True)
    var = jnp.sum((ov_all - mean) ** 2, axis=1, keepdims=True) / (n_cand - 1)
    thr = mean + jnp.sqrt(var)

    iota_ob = jax.lax.broadcasted_iota(jnp.int32, (_N_OB, 1), 0)
    iota_obr = jax.lax.broadcasted_iota(jnp.int32, (1, _N_OB), 1)

    corr_sum = jnp.float32(0.0)
    loc_sum = jnp.float32(0.0)
    n_pos = jnp.float32(0.0)

    for lv in range(len(_FMAP_DIMS)):
        s0 = _BOUNDS[lv]
        ov_cols, in_cols, idx_cols = per_level[lv]
        written = []
        for c in range(_N_CANDIDATES):
            ov_c, in_c, idx_c = ov_cols[c], in_cols[c], idx_cols[c]
            cond = (ov_c > thr) & (in_c == 1)
            score = jnp.where(cond, ov_c, -1.0)
            maxv = jnp.max(score)
            best = jnp.min(jnp.where(score == maxv, iota_ob, _N_OB))
            any_valid = jnp.max(jnp.where(cond, 1, 0)) > 0
            bh = iota_ob == best
            pidx = jnp.sum(jnp.where(bh, idx_c, 0))
            dup = jnp.bool_(False)
            for (can_j, pidx_j) in written:
                dup = dup | (can_j & (pidx_j == pidx))
            can = any_valid & jnp.logical_not(dup)
            written.append((can, pidx))

            canf = jnp.where(can, 1.0, 0.0)
            gidx = s0 + pidx

            srow = scores_ref[0, pl.ds(gidx, 1), :]
            lrow = locs_ref[0, pl.ds(gidx, 1), :]
            prow = pc_ref[pl.ds(gidx, 1), :]
            gt_row = jnp.sum(jnp.where(bh, gt_xy, 0.0),
                             axis=0, keepdims=True)
            lab = jnp.sum(jnp.where(iota_obr == best, labels_row, 0))

            cls_oh = jax.lax.broadcasted_iota(jnp.int32, (1, _N_CLASSES), 1) \
                == (lab - 1)
            s_l = jnp.sum(jnp.where(cls_oh, srow, 0.0))
            p_l = jax.nn.sigmoid(s_l)
            t1 = _pow_gamma(1.0 - p_l) * _log_sigmoid(s_l)
            t2 = _pow_gamma(p_l) * _log_sigmoid(-s_l)
            corr_sum += canf * (-_ALPHA * t1 + (1.0 - _ALPHA) * t2)

            pr_c = prow[:, 0:2]
            pr_s = prow[:, 2:4]
            dec_c = lrow[:, 0:2] * pr_s / 10.0 + pr_c
            dec_s = jnp.exp(lrow[:, 2:4] / 5.0) * pr_s
            dec = jnp.concatenate([dec_c - dec_s * 0.5,
                                   dec_c + dec_s * 0.5], axis=1)
            loc_sum += canf * jnp.sum(_smooth_l1(dec - gt_row))
            n_pos += canf

    rowi = jax.lax.broadcasted_iota(jnp.int32, (8, 128), 0)
    col = jax.lax.broadcasted_iota(jnp.int32, (8, 128), 1)
    zero = jnp.zeros((8, 128), jnp.float32)
    vals = jnp.where((rowi == 0) & (col == 0), neg_sum + corr_sum, zero)
    vals = jnp.where((rowi == 0) & (col == 1), loc_sum, vals)
    vals = jnp.where((rowi == 0) & (col == 2), n_pos, vals)
    out_ref[0] = vals


@functools.partial(jax.jit, static_argnames=())
def kernel(predicted_locs, predicted_scores, boxes, labels):
    batch = predicted_locs.shape[0]
    pc = _make_priors_cxcy()
    pxy = jnp.concatenate([pc[:, :2] - pc[:, 2:] / 2.0,
                           pc[:, :2] + pc[:, 2:] / 2.0], axis=1)
    pcT = pc.T
    pxyT = pxy.T
    labels_i = labels.astype(jnp.int32).reshape(batch, 1, _N_OB)
    boxes_f = boxes.astype(jnp.float32)

    partials = pl.pallas_call(
        _loss_body,
        grid=(batch,),
        in_specs=[
            pl.BlockSpec((1, _N_PRIORS, 4), lambda i: (i, 0, 0)),
            pl.BlockSpec((1, _N_PRIORS, _N_CLASSES), lambda i: (i, 0, 0)),
            pl.BlockSpec((1, _N_OB, 4), lambda i: (i, 0, 0)),
            pl.BlockSpec((1, 1, _N_OB), lambda i: (i, 0, 0)),
            pl.BlockSpec((4, _N_PRIORS), lambda i: (0, 0)),
            pl.BlockSpec((4, _N_PRIORS), lambda i: (0, 0)),
            pl.BlockSpec((_N_PRIORS, 4), lambda i: (0, 0)),
        ],
        out_specs=pl.BlockSpec((1, 8, 128), lambda i: (i, 0, 0)),
        out_shape=jax.ShapeDtypeStruct((batch, 8, 128), jnp.float32),
        compiler_params=pltpu.CompilerParams(
            dimension_semantics=("parallel",)),
    )(predicted_locs, predicted_scores, boxes_f, labels_i, pcT, pxyT, pc)

    conf_sum = jnp.sum(partials[:, 0, 0])
    loc_total = jnp.sum(partials[:, 0, 1])
    n_pos = jnp.sum(partials[:, 0, 2])
    conf_loss = conf_sum / jnp.maximum(n_pos, 1.0)
    loc_loss = loc_total / jnp.maximum(n_pos * 4.0, 1.0)
    return conf_loss + loc_loss


# speedup vs baseline: 1.0021x; 1.0021x over previous
import functools

import jax
import jax.numpy as jnp
import numpy as np
from jax.experimental import pallas as pl
from jax.experimental.pallas import tpu as pltpu

_FMAP_DIMS = [(100, 100), (50, 50), (25, 25), (13, 13), (7, 7)]
_OBJ_SCALES = [0.1, 0.2, 0.4, 0.6, 0.8]
_N_CANDIDATES = 11
_GAMMA = 2.5
_ALPHA = 0.25
_N_CLASSES = 8
_N_OB = 8

_SIZES = [h * w for h, w in _FMAP_DIMS]
_BOUNDS = np.concatenate([[0], np.cumsum(_SIZES)]).tolist()
_N_PRIORS = _BOUNDS[-1]


def _make_priors_cxcy():
    out = []
    for (h, w), s in zip(_FMAP_DIMS, _OBJ_SCALES):
        cy, cx = jnp.meshgrid((jnp.arange(h) + 0.5) / h,
                              (jnp.arange(w) + 0.5) / w, indexing="ij")
        p = jnp.stack([cx.reshape(-1), cy.reshape(-1),
                       jnp.full((h * w,), s, dtype=jnp.float32),
                       jnp.full((h * w,), s, dtype=jnp.float32)], axis=1)
        out.append(jnp.clip(p, 0.0, 1.0))
    return jnp.concatenate(out, axis=0)


def _log_sigmoid(x):
    return jnp.minimum(x, 0.0) - jnp.log1p(jnp.exp(-jnp.abs(x)))


def _pow_gamma(p):
    return p * p * jnp.sqrt(p)


def _smooth_l1(x):
    ax = jnp.abs(x)
    return jnp.where(ax < 1.0, 0.5 * x * x, ax - 0.5)


def _loss_body(locs_ref, scores_ref, boxes_ref, labels_ref,
               pcT_ref, pxyT_ref, pc_ref, out_ref):
    scores = scores_ref[0]
    gt_xy = boxes_ref[0]
    labels_row = labels_ref[0]

    s = scores
    ea = jnp.exp(-jnp.abs(s))
    l1p = jnp.log1p(ea)
    ls_pos = jnp.minimum(s, 0.0) - l1p
    ls_neg = jnp.minimum(-s, 0.0) - l1p
    term2 = jnp.exp(_GAMMA * ls_pos) * ls_neg
    neg_sum = -(1.0 - _ALPHA) * jnp.sum(term2)

    gt_cx = (gt_xy[:, 0:1] + gt_xy[:, 2:3]) * 0.5
    gt_cy = (gt_xy[:, 1:2] + gt_xy[:, 3:4]) * 0.5
    area_g = (gt_xy[:, 2:3] - gt_xy[:, 0:1]) * (gt_xy[:, 3:4] - gt_xy[:, 1:2])

    per_level = []
    all_ov_cols = []
    for lv in range(len(_FMAP_DIMS)):
        s0, s1 = _BOUNDS[lv], _BOUNDS[lv + 1]
        px = pcT_ref[0:1, s0:s1]
        py = pcT_ref[1:2, s0:s1]
        x1 = pxyT_ref[0:1, s0:s1]
        y1 = pxyT_ref[1:2, s0:s1]
        x2 = pxyT_ref[2:3, s0:s1]
        y2 = pxyT_ref[3:4, s0:s1]
        P = s1 - s0

        dx = gt_cx - px
        dy = gt_cy - py
        dist = dx * dx + dy * dy

        iw = jnp.clip(jnp.minimum(gt_xy[:, 2:3], x2) -
                      jnp.maximum(gt_xy[:, 0:1], x1), 0.0, None)
        ih = jnp.clip(jnp.minimum(gt_xy[:, 3:4], y2) -
                      jnp.maximum(gt_xy[:, 1:2], y1), 0.0, None)
        inter = iw * ih
        area_p = (x2 - x1) * (y2 - y1)
        ovl = inter / (area_g + area_p - inter)

        inside = jnp.where((gt_xy[:, 0:1] <= px) & (px <= gt_xy[:, 2:3]) &
                           (gt_xy[:, 1:2] <= py) & (py <= gt_xy[:, 3:4]),
                           jnp.int32(1), jnp.int32(0))

        iota = jax.lax.broadcasted_iota(jnp.int32, (_N_OB, P), 1)
        d = dist
        ov_cols, in_cols, idx_cols = [], [], []
        for _ in range(_N_CANDIDATES):
            mval = jnp.min(d, axis=1, keepdims=True)
            midx = jnp.min(jnp.where(d == mval, iota, P),
                           axis=1, keepdims=True)
            onehot = iota == midx
            ov_c = jnp.sum(jnp.where(onehot, ovl, 0.0),
                           axis=1, keepdims=True)
            in_c = jnp.max(jnp.where(onehot, inside, 0),
                           axis=1, keepdims=True)
            d = jnp.where(onehot, 1e30, d)
            ov_cols.append(ov_c)
            in_cols.append(in_c)
            idx_cols.append(midx)
        per_level.append((ov_cols, in_cols, idx_cols))
        all_ov_cols.extend(ov_cols)

    ov_all = jnp.concatenate(all_ov_cols, axis=1)
    n_cand = ov_all.shape[1]
    mean = jnp.mean(ov_all, axis=1, keepdims=True)
    var = jnp.sum((ov_all - mean) ** 2, axis=1, keepdims=True) / (n_cand - 1)
    thr = mean + jnp.sqrt(var)

    iota_ob = jax.lax.broadcasted_iota(jnp.int32, (_N_OB, 1), 0)
    iota_obr = jax.lax.broadcasted_iota(jnp.int32, (1, _N_OB), 1)

    corr_sum = jnp.float32(0.0)
    loc_sum = jnp.float32(0.0)
    n_pos = jnp.float32(0.0)

    for lv in range(len(_FMAP_DIMS)):
        s0 = _BOUNDS[lv]
        ov_cols, in_cols, idx_cols = per_level[lv]
        written = []
        for c in range(_N_CANDIDATES):
            ov_c, in_c, idx_c = ov_cols[c], in_cols[c], idx_cols[c]
            cond = (ov_c > thr) & (in_c == 1)
            score = jnp.where(cond, ov_c, -1.0)
            maxv = jnp.max(score)
            best = jnp.min(jnp.where(score == maxv, iota_ob, _N_OB))
            any_valid = jnp.max(jnp.where(cond, 1, 0)) > 0
            bh = iota_ob == best
            pidx = jnp.sum(jnp.where(bh, idx_c, 0))
            dup = jnp.bool_(False)
            for (can_j, pidx_j) in written:
                dup = dup | (can_j & (pidx_j == pidx))
            can = any_valid & jnp.logical_not(dup)
            written.append((can, pidx))

            canf = jnp.where(can, 1.0, 0.0)
            gidx = s0 + pidx

            srow = scores_ref[0, pl.ds(gidx, 1), :]
            lrow = locs_ref[0, pl.ds(gidx, 1), :]
            prow = pc_ref[pl.ds(gidx, 1), :]
            gt_row = jnp.sum(jnp.where(bh, gt_xy, 0.0),
                             axis=0, keepdims=True)
            lab = jnp.sum(jnp.where(iota_obr == best, labels_row, 0))

            cls_oh = jax.lax.broadcasted_iota(jnp.int32, (1, _N_CLASSES), 1) \
                == (lab - 1)
            s_l = jnp.sum(jnp.where(cls_oh, srow, 0.0))
            p_l = jax.nn.sigmoid(s_l)
            t1 = _pow_gamma(1.0 - p_l) * _log_sigmoid(s_l)
            t2 = _pow_gamma(p_l) * _log_sigmoid(-s_l)
            corr_sum += canf * (-_ALPHA * t1 + (1.0 - _ALPHA) * t2)

            pr_c = prow[:, 0:2]
            pr_s = prow[:, 2:4]
            dec_c = lrow[:, 0:2] * pr_s / 10.0 + pr_c
            dec_s = jnp.exp(lrow[:, 2:4] / 5.0) * pr_s
            dec = jnp.concatenate([dec_c - dec_s * 0.5,
                                   dec_c + dec_s * 0.5], axis=1)
            loc_sum += canf * jnp.sum(_smooth_l1(dec - gt_row))
            n_pos += canf

    rowi = jax.lax.broadcasted_iota(jnp.int32, (8, 128), 0)
    col = jax.lax.broadcasted_iota(jnp.int32, (8, 128), 1)
    zero = jnp.zeros((8, 128), jnp.float32)
    vals = jnp.where((rowi == 0) & (col == 0), neg_sum + corr_sum, zero)
    vals = jnp.where((rowi == 0) & (col == 1), loc_sum, vals)
    vals = jnp.where((rowi == 0) & (col == 2), n_pos, vals)
    out_ref[0] = vals


@functools.partial(jax.jit, static_argnames=())
def kernel(predicted_locs, predicted_scores, boxes, labels):
    batch = predicted_locs.shape[0]
    pc = _make_priors_cxcy()
    pxy = jnp.concatenate([pc[:, :2] - pc[:, 2:] / 2.0,
                           pc[:, :2] + pc[:, 2:] / 2.0], axis=1)
    pcT = pc.T
    pxyT = pxy.T
    labels_i = labels.astype(jnp.int32).reshape(batch, 1, _N_OB)
    boxes_f = boxes.astype(jnp.float32)

    partials = pl.pallas_call(
        _loss_body,
        grid=(batch,),
        in_specs=[
            pl.BlockSpec((1, _N_PRIORS, 4), lambda i: (i, 0, 0)),
            pl.BlockSpec((1, _N_PRIORS, _N_CLASSES), lambda i: (i, 0, 0)),
            pl.BlockSpec((1, _N_OB, 4), lambda i: (i, 0, 0)),
            pl.BlockSpec((1, 1, _N_OB), lambda i: (i, 0, 0)),
            pl.BlockSpec((4, _N_PRIORS), lambda i: (0, 0)),
            pl.BlockSpec((4, _N_PRIORS), lambda i: (0, 0)),
            pl.BlockSpec((_N_PRIORS, 4), lambda i: (0, 0)),
        ],
        out_specs=pl.BlockSpec((1, 8, 128), lambda i: (i, 0, 0)),
        out_shape=jax.ShapeDtypeStruct((batch, 8, 128), jnp.float32),
        compiler_params=pltpu.CompilerParams(
            dimension_semantics=("parallel",)),
    )(predicted_locs, predicted_scores, boxes_f, labels_i, pcT, pxyT, pc)

    conf_sum = jnp.sum(partials[:, 0, 0])
    loc_total = jnp.sum(partials[:, 0, 1])
    n_pos = jnp.sum(partials[:, 0, 2])
    conf_loss = conf_sum / jnp.maximum(n_pos, 1.0)
    loc_loss = loc_total / jnp.maximum(n_pos * 4.0, 1.0)
    return conf_loss + loc_loss
